# baseline (speedup 1.0000x reference)
import jax
import jax.numpy as jnp
from jax.experimental import pallas as pl

B, T, D = 4096, 200, 64
_BB = 128


def _body(x_ref, o_ref):
    o_ref[...] = x_ref[0:1, 0:8, :]


def kernel(x, ticker, embed):
    return pl.pallas_call(
        _body,
        grid=(B // _BB,),
        in_specs=[pl.BlockSpec((_BB, T, D), lambda i: (i, 0, 0))],
        out_specs=pl.BlockSpec((1, 8, D), lambda i: (i, 0, 0)),
        out_shape=jax.ShapeDtypeStruct((B // _BB, 8, D), jnp.float32),
    )(x)


# speedup vs baseline: 1.0003x; 1.0003x over previous
import jax
import jax.numpy as jnp
from jax.experimental import pallas as pl

B, T, D = 4096, 200, 64
_BB = 128
_K = 8
_SB = _BB // _K


def _body(*refs):
    o_ref = refs[-1]
    acc = jnp.zeros((), jnp.float32)
    for r in refs[:-1]:
        acc = acc + r[0, 0, 0]
    o_ref[...] = jnp.full((1, 8, D), acc, jnp.float32)


def kernel(x, ticker, embed):
    def mk(k):
        return pl.BlockSpec((_SB, T, D), lambda i, k=k: (i * _K + k, 0, 0))

    return pl.pallas_call(
        _body,
        grid=(B // _BB,),
        in_specs=[mk(k) for k in range(_K)],
        out_specs=pl.BlockSpec((1, 8, D), lambda i: (i, 0, 0)),
        out_shape=jax.ShapeDtypeStruct((B // _BB, 8, D), jnp.float32),
    )(*([x] * _K))
